# Initial kernel scaffold; baseline (speedup 1.0000x reference)
import jax
import jax.numpy as jnp
from jax.experimental import pallas as pl


def kernel(pos, batch):
    raise NotImplementedError("write your pallas kernel here")


# speedup vs baseline: 7.4734x; 7.4734x over previous
import functools

import jax
import jax.numpy as jnp
from jax import lax
from jax.experimental import pallas as pl
from jax.experimental.pallas import tpu as pltpu
from jax.experimental.pallas import tpu_sc as plsc

N = 65536
M = 4096
NS = 16
PPT = N // NS
VPT = PPT // 16
REC = 64
BIG = 1 << 30


def _fps_body(x_hbm, y_hbm, z_hbm, out_hbm,
              xs, ys, zs, ds, lv, stage, oidx, pub):
    s = lax.axis_index("s")
    base = s * PPT
    lane = lax.iota(jnp.int32, 16)
    shift = PPT.bit_length() - 1

    pltpu.sync_copy(x_hbm.at[pl.ds(base, PPT)], xs)
    pltpu.sync_copy(y_hbm.at[pl.ds(base, PPT)], ys)
    pltpu.sync_copy(z_hbm.at[pl.ds(base, PPT)], zs)

    inf16 = jnp.full((16,), jnp.inf, jnp.float32)

    def init_v(v, c):
        ds[pl.ds(v * 16, 16)] = inf16
        return c

    lax.fori_loop(0, VPT, init_v, 0)

    def publish(val_row, bx, by, bz):
        stage[pl.ds(0, 16)] = val_row
        stage[pl.ds(16, 16)] = bx
        stage[pl.ds(32, 16)] = by
        stage[pl.ds(48, 16)] = bz
        pltpu.sync_copy(stage, pub.at[pl.ds(s * REC, REC)])

    sv = jnp.full((16,), s, jnp.int32)
    v0 = jnp.where(s == 0, 1.0, 0.0).astype(jnp.float32)
    row0 = jnp.where(lane == 0, v0, 0.0)
    publish(row0, plsc.load_gather(xs, [sv]), plsc.load_gather(ys, [sv]),
            plsc.load_gather(zs, [sv]))
    plsc.subcore_barrier()

    lane_rec = lane * REC

    def outer(i, carry):
        pltpu.sync_copy(pub, lv)
        plsc.subcore_barrier()
        vals = plsc.load_gather(lv, [lane_rec])
        idxf = plsc.load_gather(lv, [lane_rec + 1])
        gval = jnp.max(vals)
        gidx = jnp.min(jnp.where(vals == gval, idxf.astype(jnp.int32), BIG))
        wb = (gidx >> shift) * REC
        cx = lv[pl.ds(wb + 16, 16)]
        cy = lv[pl.ds(wb + 32, 16)]
        cz = lv[pl.ds(wb + 48, 16)]
        plsc.store_scatter(oidx, [jnp.full((16,), i, jnp.int32)],
                           jnp.full((16,), gidx, jnp.int32), mask=lane == 0)

        def inner(v, st):
            rm, ri = st
            sl = pl.ds(v * 16, 16)
            dxv = xs[sl] - cx
            dyv = ys[sl] - cy
            dzv = zs[sl] - cz
            d = dxv * dxv + dyv * dyv + dzv * dzv
            nd = jnp.minimum(ds[sl], d)
            ds[sl] = nd
            upd = nd > rm
            ri = jnp.where(upd, lane + v * 16, ri)
            rm = jnp.maximum(rm, nd)
            return rm, ri

        rm, ri = lax.fori_loop(
            0, VPT, inner,
            (jnp.full((16,), -jnp.inf, jnp.float32), jnp.zeros((16,), jnp.int32)))

        mv = jnp.max(rm)
        lix = jnp.min(jnp.where(rm == mv, ri, BIG))
        ivec = jnp.full((16,), lix, jnp.int32)
        gixf = (base + lix).astype(jnp.float32)
        row = jnp.where(lane == 0, mv, jnp.where(lane == 1, gixf, 0.0))
        publish(row, plsc.load_gather(xs, [ivec]), plsc.load_gather(ys, [ivec]),
                plsc.load_gather(zs, [ivec]))
        plsc.subcore_barrier()
        return carry

    lax.fori_loop(0, M, outer, 0)

    @pl.when(s == 0)
    def _():
        pltpu.sync_copy(oidx, out_hbm)


@functools.partial(jax.jit, static_argnums=())
def _fps(x, y, z):
    mesh = plsc.VectorSubcoreMesh(
        core_axis_name="c", subcore_axis_name="s", num_cores=1)
    f = pl.kernel(
        _fps_body,
        out_type=jax.ShapeDtypeStruct((M,), jnp.int32),
        mesh=mesh,
        compiler_params=pltpu.CompilerParams(needs_layout_passes=False),
        scratch_types=[
            pltpu.VMEM((PPT,), jnp.float32),
            pltpu.VMEM((PPT,), jnp.float32),
            pltpu.VMEM((PPT,), jnp.float32),
            pltpu.VMEM((PPT,), jnp.float32),
            pltpu.VMEM((NS * REC,), jnp.float32),
            pltpu.VMEM((REC,), jnp.float32),
            pltpu.VMEM((M,), jnp.int32),
            pltpu.VMEM_SHARED((NS * REC,), jnp.float32),
        ],
    )
    return f(x, y, z)


def kernel(pos, batch):
    del batch
    posT = pos.T.reshape(3, N)
    return _fps(posT[0], posT[1], posT[2])


# speedup vs baseline: 21.5018x; 2.8771x over previous
import functools

import jax
import jax.numpy as jnp
from jax import lax
from jax.experimental import pallas as pl
from jax.experimental.pallas import tpu as pltpu
from jax.experimental.pallas import tpu_sc as plsc

N = 65536
M = 4096
NS = 16
PPT = N // NS
VPT = PPT // 16
REC = 64
BIG = 1 << 30


def _fps_body(x_hbm, y_hbm, z_hbm, out_hbm,
              xs, ys, zs, ds, lv, stage, oidx, pub):
    s = lax.axis_index("s")
    base = s * PPT
    lane = lax.iota(jnp.int32, 16)
    shift = PPT.bit_length() - 1

    pltpu.sync_copy(x_hbm.at[pl.ds(base, PPT)], xs)
    pltpu.sync_copy(y_hbm.at[pl.ds(base, PPT)], ys)
    pltpu.sync_copy(z_hbm.at[pl.ds(base, PPT)], zs)

    inf16 = jnp.full((16,), jnp.inf, jnp.float32)

    def init_v(v, c):
        ds[pl.ds(v * 16, 16)] = inf16
        return c

    lax.fori_loop(0, VPT, init_v, 0)

    def publish(val_row, bx, by, bz):
        stage[pl.ds(0, 16)] = val_row
        stage[pl.ds(16, 16)] = bx
        stage[pl.ds(32, 16)] = by
        stage[pl.ds(48, 16)] = bz
        pltpu.sync_copy(stage, pub.at[pl.ds(s * REC, REC)])

    sv = jnp.full((16,), s, jnp.int32)
    v0 = jnp.where(s == 0, 1.0, 0.0).astype(jnp.float32)
    row0 = jnp.where(lane == 0, v0, 0.0)
    publish(row0, plsc.load_gather(xs, [sv]), plsc.load_gather(ys, [sv]),
            plsc.load_gather(zs, [sv]))
    plsc.subcore_barrier()

    lane_rec = lane * REC

    def outer(i, carry):
        pltpu.sync_copy(pub, lv)
        plsc.subcore_barrier()
        vals = plsc.load_gather(lv, [lane_rec])
        idxf = plsc.load_gather(lv, [lane_rec + 1])
        gval = jnp.max(vals)
        gidx = jnp.min(jnp.where(vals == gval, idxf.astype(jnp.int32), BIG))
        wb = (gidx >> shift) * REC
        cx = lv[pl.ds(wb + 16, 16)]
        cy = lv[pl.ds(wb + 32, 16)]
        cz = lv[pl.ds(wb + 48, 16)]
        plsc.store_scatter(oidx, [jnp.full((16,), i, jnp.int32)],
                           jnp.full((16,), gidx, jnp.int32), mask=lane == 0)

        @plsc.parallel_loop(
            0, VPT, unroll=8,
            carry=(jnp.full((16,), -jnp.inf, jnp.float32),
                   jnp.zeros((16,), jnp.int32)))
        def inner(v, st):
            rm, ri = st
            sl = pl.ds(v * 16, 16)
            dxv = xs[sl] - cx
            dyv = ys[sl] - cy
            dzv = zs[sl] - cz
            d = dxv * dxv + dyv * dyv + dzv * dzv
            nd = jnp.minimum(ds[sl], d)
            ds[sl] = nd
            upd = nd > rm
            ri = jnp.where(upd, v, ri)
            rm = jnp.maximum(rm, nd)
            return rm, ri

        rm, ri = inner

        mv = jnp.max(rm)
        lix = jnp.min(jnp.where(rm == mv, ri * 16 + lane, BIG))
        ivec = jnp.full((16,), lix, jnp.int32)
        gixf = (base + lix).astype(jnp.float32)
        row = jnp.where(lane == 0, mv, jnp.where(lane == 1, gixf, 0.0))
        publish(row, plsc.load_gather(xs, [ivec]), plsc.load_gather(ys, [ivec]),
                plsc.load_gather(zs, [ivec]))
        plsc.subcore_barrier()
        return carry

    lax.fori_loop(0, M, outer, 0)

    @pl.when(s == 0)
    def _():
        pltpu.sync_copy(oidx, out_hbm)


@functools.partial(jax.jit, static_argnums=())
def _fps(x, y, z):
    mesh = plsc.VectorSubcoreMesh(
        core_axis_name="c", subcore_axis_name="s", num_cores=1)
    f = pl.kernel(
        _fps_body,
        out_type=jax.ShapeDtypeStruct((M,), jnp.int32),
        mesh=mesh,
        compiler_params=pltpu.CompilerParams(needs_layout_passes=False),
        scratch_types=[
            pltpu.VMEM((PPT,), jnp.float32),
            pltpu.VMEM((PPT,), jnp.float32),
            pltpu.VMEM((PPT,), jnp.float32),
            pltpu.VMEM((PPT,), jnp.float32),
            pltpu.VMEM((NS * REC,), jnp.float32),
            pltpu.VMEM((REC,), jnp.float32),
            pltpu.VMEM((M,), jnp.int32),
            pltpu.VMEM_SHARED((NS * REC,), jnp.float32),
        ],
    )
    return f(x, y, z)


def kernel(pos, batch):
    del batch
    posT = pos.T.reshape(3, N)
    return _fps(posT[0], posT[1], posT[2])


# speedup vs baseline: 22.7827x; 1.0596x over previous
import functools

import jax
import jax.numpy as jnp
from jax import lax
from jax.experimental import pallas as pl
from jax.experimental.pallas import tpu as pltpu
from jax.experimental.pallas import tpu_sc as plsc

N = 65536
M = 4096
NS = 16
PPT = N // NS
VPT = PPT // 16
BLK = NS * 16
BIG = 1 << 30


def _fps_body(x_hbm, y_hbm, z_hbm, out_hbm,
              xs, ys, zs, ds, lv, stage, oidx, pub):
    s = lax.axis_index("s")
    base = s * PPT
    lane = lax.iota(jnp.int32, 16)
    shift = PPT.bit_length() - 1

    pltpu.sync_copy(x_hbm.at[pl.ds(base, PPT)], xs)
    pltpu.sync_copy(y_hbm.at[pl.ds(base, PPT)], ys)
    pltpu.sync_copy(z_hbm.at[pl.ds(base, PPT)], zs)

    inf16 = jnp.full((16,), jnp.inf, jnp.float32)

    @plsc.parallel_loop(0, VPT, unroll=8)
    def _init(v):
        ds[pl.ds(v * 16, 16)] = inf16

    def publish(blk, mv, gixf, bx, by, bz):
        row = jnp.where(lane == 0, mv,
                        jnp.where(lane == 1, gixf,
                                  jnp.where(lane == 2, bx,
                                            jnp.where(lane == 3, by, bz))))
        stage[...] = row
        pltpu.sync_copy(stage, pub.at[pl.ds(blk * BLK + s * 16, 16)])

    sv = jnp.full((16,), s, jnp.int32)
    v0 = jnp.where(s == 0, 1.0, 0.0).astype(jnp.float32)
    publish(0, v0, jnp.float32(0.0), plsc.load_gather(xs, [sv]),
            plsc.load_gather(ys, [sv]), plsc.load_gather(zs, [sv]))
    plsc.subcore_barrier()

    lane16 = lane * 16

    def outer(i, carry):
        blk = i & 1
        pltpu.sync_copy(pub.at[pl.ds(blk * BLK, BLK)], lv)
        vals = plsc.load_gather(lv, [lane16])
        idxf = plsc.load_gather(lv, [lane16 + 1])
        gval = jnp.max(vals)
        gidx = jnp.min(jnp.where(vals == gval, idxf.astype(jnp.int32), BIG))
        wr = (gidx >> shift) * 16
        cx = plsc.load_gather(lv, [jnp.full((16,), wr + 2, jnp.int32)])
        cy = plsc.load_gather(lv, [jnp.full((16,), wr + 3, jnp.int32)])
        cz = plsc.load_gather(lv, [jnp.full((16,), wr + 4, jnp.int32)])
        plsc.store_scatter(oidx, [jnp.full((16,), i, jnp.int32)],
                           jnp.full((16,), gidx, jnp.int32), mask=lane == 0)

        @plsc.parallel_loop(
            0, VPT, unroll=16,
            carry=(jnp.full((16,), -jnp.inf, jnp.float32),
                   jnp.zeros((16,), jnp.int32)))
        def inner(v, st):
            rm, ri = st
            sl = pl.ds(v * 16, 16)
            dxv = xs[sl] - cx
            dyv = ys[sl] - cy
            dzv = zs[sl] - cz
            d = dxv * dxv + dyv * dyv + dzv * dzv
            nd = jnp.minimum(ds[sl], d)
            ds[sl] = nd
            upd = nd > rm
            ri = jnp.where(upd, v, ri)
            rm = jnp.maximum(rm, nd)
            return rm, ri

        rm, ri = inner

        mv = jnp.max(rm)
        lix = jnp.min(jnp.where(rm == mv, ri * 16 + lane, BIG))
        ivec = jnp.full((16,), lix, jnp.int32)
        gixf = (base + lix).astype(jnp.float32)
        publish(1 - blk, mv, gixf, plsc.load_gather(xs, [ivec]),
                plsc.load_gather(ys, [ivec]), plsc.load_gather(zs, [ivec]))
        plsc.subcore_barrier()
        return carry

    lax.fori_loop(0, M, outer, 0)

    @pl.when(s == 0)
    def _():
        pltpu.sync_copy(oidx, out_hbm)


@functools.partial(jax.jit, static_argnums=())
def _fps(x, y, z):
    mesh = plsc.VectorSubcoreMesh(
        core_axis_name="c", subcore_axis_name="s", num_cores=1)
    f = pl.kernel(
        _fps_body,
        out_type=jax.ShapeDtypeStruct((M,), jnp.int32),
        mesh=mesh,
        compiler_params=pltpu.CompilerParams(needs_layout_passes=False),
        scratch_types=[
            pltpu.VMEM((PPT,), jnp.float32),
            pltpu.VMEM((PPT,), jnp.float32),
            pltpu.VMEM((PPT,), jnp.float32),
            pltpu.VMEM((PPT,), jnp.float32),
            pltpu.VMEM((BLK,), jnp.float32),
            pltpu.VMEM((16,), jnp.float32),
            pltpu.VMEM((M,), jnp.int32),
            pltpu.VMEM_SHARED((2 * BLK,), jnp.float32),
        ],
    )
    return f(x, y, z)


def kernel(pos, batch):
    del batch
    posT = pos.T.reshape(3, N)
    return _fps(posT[0], posT[1], posT[2])
